# Initial kernel scaffold; baseline (speedup 1.0000x reference)
import jax
import jax.numpy as jnp
from jax.experimental import pallas as pl


def kernel(x, mask, router_W, expert_W, expert_b):
    raise NotImplementedError("write your pallas kernel here")


# speedup vs baseline: 1.0642x; 1.0642x over previous
import functools

import jax
import jax.numpy as jnp
from jax import lax
from jax.experimental import pallas as pl
from jax.experimental.pallas import tpu as pltpu
from jax.experimental.pallas import tpu_sc as plsc


def _pool_body(mask_ref, rw_ref, x_ref, pooled_ref, logits_ref, cnt_ref, *,
               t_blocks):
    t = pl.program_id(1)

    @pl.when(t == 0)
    def _init():
        pooled_ref[...] = jnp.zeros_like(pooled_ref)
        cnt_ref[...] = jnp.zeros_like(cnt_ref)

    mask_f = mask_ref[...].astype(jnp.float32)
    x = x_ref[...]
    pooled_ref[...] += jnp.sum(x * mask_f[:, :, None], axis=1)
    cnt_ref[...] += jnp.sum(mask_f, axis=1, keepdims=True)

    @pl.when(t == t_blocks - 1)
    def _fin():
        denom = jnp.maximum(cnt_ref[:, 0:1], 1.0)
        pooled = pooled_ref[...] / denom
        pooled_ref[...] = pooled
        logits_ref[...] = lax.dot_general(
            pooled, rw_ref[...], (((1,), (1,)), ((), ())),
            preferred_element_type=jnp.float32)


def _pool_and_route(x, mask, router_W):
    B, T, D = x.shape
    E = router_W.shape[0]
    BB, TB = 8, 128
    grid = (B // BB, T // TB)
    return pl.pallas_call(
        functools.partial(_pool_body, t_blocks=grid[1]),
        grid=grid,
        in_specs=[
            pl.BlockSpec((BB, TB), lambda b, t: (b, t)),
            pl.BlockSpec((E, D), lambda b, t: (0, 0)),
            pl.BlockSpec((BB, TB, D), lambda b, t: (b, t, 0)),
        ],
        scratch_shapes=[pltpu.VMEM((BB, 128), jnp.float32)],
        out_specs=[
            pl.BlockSpec((BB, D), lambda b, t: (b, 0)),
            pl.BlockSpec((BB, E), lambda b, t: (b, 0)),
        ],
        out_shape=[
            jax.ShapeDtypeStruct((B, D), jnp.float32),
            jax.ShapeDtypeStruct((B, E), jnp.float32),
        ],
        compiler_params=pltpu.CompilerParams(
            dimension_semantics=("parallel", "arbitrary")),
    )(mask, router_W, x)


def _lane_perm(v, idx):
    return lax.gather(
        v, idx[:, None],
        lax.GatherDimensionNumbers(
            offset_dims=(), collapsed_slice_dims=(0,), start_index_map=(0,)),
        slice_sizes=(1,),
        mode=lax.GatherScatterMode.PROMISE_IN_BOUNDS)


def _butterfly(v, iota, op):
    for k in (1, 2, 4, 8):
        v = op(v, _lane_perm(v, iota ^ k))
    return v


def _make_gates_kernel(B, E):
    info = plsc.get_sparse_core_info()
    nw = info.num_cores * info.num_subcores
    rows = B // nw

    @functools.partial(
        pl.kernel,
        mesh=plsc.VectorSubcoreMesh(core_axis_name="c", subcore_axis_name="s"),
        out_type=jax.ShapeDtypeStruct((B, E), jnp.float32),
        scratch_types=[
            pltpu.VMEM((rows, E), jnp.float32),
            pltpu.VMEM((rows, E), jnp.float32),
        ],
    )
    def gates_kernel(logits_hbm, out_hbm, in_v, out_v):
        wid = lax.axis_index("s") * info.num_cores + lax.axis_index("c")
        base = wid * rows
        pltpu.sync_copy(logits_hbm.at[pl.ds(base, rows)], in_v)
        iota = lax.iota(jnp.int32, E)
        big = jnp.int32(E)
        for i in range(rows):
            row = in_v[i, :]
            m = _butterfly(row, iota, jnp.maximum)
            p = jnp.exp(row - m)
            z = _butterfly(p, iota, jnp.add)
            probs = p / z
            v1 = _butterfly(probs, iota, jnp.maximum)
            i1 = _butterfly(jnp.where(probs == v1, iota, big), iota,
                            jnp.minimum)
            rest = jnp.where(iota == i1, -1.0, probs)
            v2 = _butterfly(rest, iota, jnp.maximum)
            i2 = _butterfly(jnp.where(rest == v2, iota, big), iota,
                            jnp.minimum)
            denom = v1 + v2 + 1e-9
            g = jnp.where(iota == i1, v1 / denom,
                          jnp.where(iota == i2, v2 / denom, 0.0))
            out_v[i, :] = g
        pltpu.sync_copy(out_v, out_hbm.at[pl.ds(base, rows)])

    return gates_kernel


def _expert_body(gates_ref, b_ref, pooled_ref, w_ref, out_ref):
    e = pl.program_id(0)
    gates = gates_ref[...]

    @pl.when(e == 0)
    def _init():
        out_ref[...] = jnp.dot(gates, b_ref[...],
                               preferred_element_type=jnp.float32)

    E = gates.shape[1]
    sel = (lax.broadcasted_iota(jnp.int32, gates.shape, 1) == e)
    col = jnp.sum(jnp.where(sel, gates, 0.0), axis=1, keepdims=True)
    y = lax.dot_general(
        pooled_ref[...], w_ref[0],
        (((1,), (1,)), ((), ())),
        preferred_element_type=jnp.float32)
    out_ref[...] += y * col


def _expert_combine(gates, expert_b, pooled, expert_W):
    E, C, D = expert_W.shape
    B = pooled.shape[0]
    return pl.pallas_call(
        _expert_body,
        grid=(E,),
        in_specs=[
            pl.BlockSpec((B, E), lambda e: (0, 0)),
            pl.BlockSpec((E, C), lambda e: (0, 0)),
            pl.BlockSpec((B, D), lambda e: (0, 0)),
            pl.BlockSpec((1, C, D), lambda e: (e, 0, 0)),
        ],
        out_specs=pl.BlockSpec((B, C), lambda e: (0, 0)),
        out_shape=jax.ShapeDtypeStruct((B, C), jnp.float32),
        compiler_params=pltpu.CompilerParams(
            dimension_semantics=("arbitrary",)),
    )(gates, expert_b, pooled, expert_W)


def kernel(x, mask, router_W, expert_W, expert_b):
    pooled, logits = _pool_and_route(x, mask, router_W)
    gates = _make_gates_kernel(*logits.shape)(logits)
    return _expert_combine(gates, expert_b, pooled, expert_W)
